# baseline (speedup 1.0000x reference)
import functools

import jax
import jax.numpy as jnp
from jax import lax
from jax.experimental import pallas as pl
from jax.experimental.pallas import tpu as pltpu
from jax.experimental.pallas import tpu_sc as plsc

_N = 100000
_NC = 32
_FC = 64
_E = 1600000
_HALF = 50000
_ACC_ROWS = 50048
_EP_ROWS = 12800
_R_SUB = 800
_C = 2
_ITERS = _R_SUB // _C
_ZROWS = _ACC_ROWS // 16


def _shift_body(x_hbm, src_hbm, dst_hbm, zeros_hbm, out_hbm,
                acc, src_a, dst_a, rows_a, src_b, dst_b, rows_b, gsem, ssem):
    c = lax.axis_index("c")
    s = lax.axis_index("s")
    lo = c * _HALF
    hi = lo + _HALF
    gvec = jnp.arange(16, dtype=jnp.int32) + (_HALF + 16 * (s % 3))

    pltpu.sync_copy(zeros_hbm, rows_a)
    z0 = s * _ZROWS
    for k in range(12):
        pltpu.sync_copy(rows_a, acc.at[pl.ds(z0 + k * 256, 256)])
    pltpu.sync_copy(rows_a.at[pl.ds(0, 56)], acc.at[pl.ds(z0 + 3072, 56)])
    plsc.subcore_barrier()

    base_row = s * _R_SUB

    def _load_idx(g, sb, db):
        row = base_row + g * _C
        pltpu.sync_copy(src_hbm.at[pl.ds(row, _C)], sb)
        pltpu.sync_copy(dst_hbm.at[pl.ds(row, _C)], db)

    def _remap(db):
        for j in range(_C):
            for v in range(8):
                d = db[j, pl.ds(v * 16, 16)]
                inr = (d >= lo) & (d < hi)
                db[j, pl.ds(v * 16, 16)] = jnp.where(inr, d - lo, gvec)

    def _fire_gathers(sb, rb):
        for j in range(_C):
            pltpu.async_copy(x_hbm.at[sb.at[j]],
                             rb.at[pl.ds(j * 128, 128)], gsem)

    def _drain_gathers(sb, rb):
        for j in range(_C):
            pltpu.make_async_copy(x_hbm.at[sb.at[j]],
                                  rb.at[pl.ds(j * 128, 128)], gsem).wait()

    def _fire_scatters(rb, db):
        for j in range(_C):
            pltpu.async_copy(rb.at[pl.ds(j * 128, 128)],
                             acc.at[db.at[j]], ssem, add=True)

    def _drain_scatters(rb, db):
        for j in range(_C):
            pltpu.make_async_copy(rb.at[pl.ds(j * 128, 128)],
                                  acc.at[db.at[j]], ssem).wait()

    bufs = ((src_a, dst_a, rows_a), (src_b, dst_b, rows_b))

    _load_idx(0, src_a, dst_a)
    _fire_gathers(src_a, rows_a)
    _load_idx(1, src_b, dst_b)
    _remap(dst_a)
    _drain_gathers(src_a, rows_a)
    _fire_gathers(src_b, rows_b)
    _fire_scatters(rows_a, dst_a)

    def _pair(k, carry):
        g = 1 + 2 * k
        for i, (cur, nxt) in enumerate(((bufs[1], bufs[0]),
                                        (bufs[0], bufs[1]))):
            cs, cd, cr = cur
            ns, nd, nr = nxt
            gg = g + i
            _remap(cd)
            _drain_scatters(nr, nd)
            _load_idx(gg + 1, ns, nd)
            _drain_gathers(cs, cr)
            _fire_gathers(ns, nr)
            _fire_scatters(cr, cd)
        return carry

    lax.fori_loop(0, (_ITERS - 2) // 2, _pair, 0)

    _remap(dst_b)
    _drain_scatters(rows_a, dst_a)
    _drain_gathers(src_b, rows_b)
    _fire_scatters(rows_b, dst_b)
    _drain_scatters(rows_b, dst_b)
    plsc.subcore_barrier()

    a0 = s * 3128

    @pl.when(s < 15)
    def _wb_main():
        pltpu.sync_copy(acc.at[pl.ds(a0, 3128)],
                        out_hbm.at[pl.ds(c * _HALF + a0, 3128)])

    @pl.when(s == 15)
    def _wb_tail():
        pltpu.sync_copy(acc.at[pl.ds(a0, 3080)],
                        out_hbm.at[pl.ds(c * _HALF + a0, 3080)])


@functools.cache
def _shift_call():
    return pl.kernel(
        _shift_body,
        out_type=jax.ShapeDtypeStruct((_N, _FC), jnp.bfloat16),
        mesh=plsc.VectorSubcoreMesh(core_axis_name="c", subcore_axis_name="s"),
        compiler_params=pltpu.CompilerParams(use_tc_tiling_on_sc=False),
        scratch_types=[
            pltpu.VMEM_SHARED((_ACC_ROWS, _FC), jnp.bfloat16),
            pltpu.VMEM((_C, 128), jnp.int32),
            pltpu.VMEM((_C, 128), jnp.int32),
            pltpu.VMEM((_C * 128, _FC), jnp.bfloat16),
            pltpu.VMEM((_C, 128), jnp.int32),
            pltpu.VMEM((_C, 128), jnp.int32),
            pltpu.VMEM((_C * 128, _FC), jnp.bfloat16),
            pltpu.SemaphoreType.DMA,
            pltpu.SemaphoreType.DMA,
        ],
    )


def _mlp_kernel(col, x_ref, w1_ref, b1_ref, w2_ref, b2_ref, o_ref):
    x = x_ref[...]
    if col is not None:
        x = x[:, col:col + _NC]
    x = x.astype(jnp.float32)
    h = jnp.dot(x, w1_ref[...], preferred_element_type=jnp.float32)
    h = jnp.maximum(h + b1_ref[...], 0.0)
    o_ref[...] = jnp.dot(h, w2_ref[...],
                         preferred_element_type=jnp.float32) + b2_ref[...]


def _mlp_res_kernel(col, x_ref, r_ref, w1_ref, b1_ref, w2_ref, b2_ref, o_ref):
    x = x_ref[...]
    if col is not None:
        x = x[:, col:col + _NC]
    x = x.astype(jnp.float32)
    h = jnp.dot(x, w1_ref[...], preferred_element_type=jnp.float32)
    h = jnp.maximum(h + b1_ref[...], 0.0)
    o_ref[...] = (r_ref[...] + jnp.dot(h, w2_ref[...],
                                       preferred_element_type=jnp.float32)
                  + b2_ref[...])


_ROW_BLK = 2000


def _mlp(x, p, residual=None, col=None):
    n, d_full = x.shape
    d_in = d_full if col is None else _NC
    hdim = p["W1"].shape[1]
    d_out = p["W2"].shape[1]
    b1 = p["b1"].reshape(1, hdim)
    b2 = p["b2"].reshape(1, d_out)
    grid = (n // _ROW_BLK,)
    x_spec = pl.BlockSpec((_ROW_BLK, d_full), lambda i: (i, 0))
    w1_spec = pl.BlockSpec((d_in, hdim), lambda i: (0, 0))
    b1_spec = pl.BlockSpec((1, hdim), lambda i: (0, 0))
    w2_spec = pl.BlockSpec((hdim, d_out), lambda i: (0, 0))
    b2_spec = pl.BlockSpec((1, d_out), lambda i: (0, 0))
    o_spec = pl.BlockSpec((_ROW_BLK, d_out), lambda i: (i, 0))
    out_shape = jax.ShapeDtypeStruct((n, d_out), jnp.float32)
    if residual is None:
        return pl.pallas_call(
            functools.partial(_mlp_kernel, col), grid=grid,
            in_specs=[x_spec, w1_spec, b1_spec, w2_spec, b2_spec],
            out_specs=o_spec, out_shape=out_shape,
        )(x, p["W1"], b1, p["W2"], b2)
    r_spec = pl.BlockSpec((_ROW_BLK, d_out), lambda i: (i, 0))
    return pl.pallas_call(
        functools.partial(_mlp_res_kernel, col), grid=grid,
        in_specs=[x_spec, r_spec, w1_spec, b1_spec, w2_spec, b2_spec],
        out_specs=o_spec, out_shape=out_shape,
    )(x, residual, p["W1"], b1, p["W2"], b2)


def kernel(f, g, edge_index, params):
    src = edge_index[0]
    dst = edge_index[1]
    pad = _EP_ROWS * 128 - _E
    src_p = jnp.concatenate(
        [src, jnp.zeros((pad,), jnp.int32)]).reshape(_EP_ROWS, 128)
    dst_p = jnp.concatenate(
        [dst, jnp.full((pad,), _N, jnp.int32)]).reshape(_EP_ROWS, 128)
    zeros_stage = jnp.zeros((_C * 128, _FC), jnp.bfloat16)

    f1 = _mlp(f, params["readin_f"])
    g1 = _mlp(g, params["readin_g"])
    for l in range(2):
        xf = jnp.concatenate([f1, g1], axis=1).astype(jnp.bfloat16)
        sh = _shift_call()(xf, src_p, dst_p, zeros_stage)
        f1n = _mlp(sh, params["convs"][l]["equi"], residual=f1, col=_NC)
        g1n = _mlp(sh, params["convs"][l]["inv"], residual=g1, col=0)
        f1, g1 = f1n, g1n
    return (_mlp(f1, params["readout_f"]), _mlp(g1, params["readout_g"]))


# speedup vs baseline: 1.1634x; 1.1634x over previous
import functools

import jax
import jax.numpy as jnp
from jax import lax
from jax.experimental import pallas as pl
from jax.experimental.pallas import tpu as pltpu
from jax.experimental.pallas import tpu_sc as plsc

_N = 100000
_NC = 32
_E = 1600000
_HALF = 50000
_ACC_ROWS = 50048
_EP_ROWS = 12800
_R_SUB = 800
_C = 8
_ITERS = _R_SUB // _C
_ZROWS = _ACC_ROWS // 16


def _shift_body(x_hbm, src_hbm, dst_hbm, zeros_hbm, out_hbm,
                acc, src_a, dst_a, rows_a, src_b, dst_b, rows_b, gsem, ssem):
    c = lax.axis_index("c")
    s = lax.axis_index("s")
    lo = c * _HALF
    hi = lo + _HALF
    gvec = jnp.arange(16, dtype=jnp.int32) + (_HALF + 16 * (s % 3))

    pltpu.sync_copy(zeros_hbm, rows_a)
    z0 = s * _ZROWS
    for k in range(3):
        pltpu.sync_copy(rows_a, acc.at[pl.ds(z0 + k * 1024, 1024)])
    pltpu.sync_copy(rows_a.at[pl.ds(0, 56)], acc.at[pl.ds(z0 + 3072, 56)])
    plsc.subcore_barrier()

    base_row = s * _R_SUB

    def _load_idx(g, sb, db):
        row = base_row + g * _C
        pltpu.sync_copy(src_hbm.at[pl.ds(row, _C)], sb)
        pltpu.sync_copy(dst_hbm.at[pl.ds(row, _C)], db)

    def _remap(db):
        for j in range(_C):
            for v in range(8):
                d = db[j, pl.ds(v * 16, 16)]
                inr = (d >= lo) & (d < hi)
                db[j, pl.ds(v * 16, 16)] = jnp.where(inr, d - lo, gvec)

    def _fire_gathers(sb, rb):
        for j in range(_C):
            pltpu.async_copy(x_hbm.at[sb.at[j]],
                             rb.at[pl.ds(j * 128, 128)], gsem)

    def _drain_gathers(sb, rb):
        for j in range(_C):
            pltpu.make_async_copy(x_hbm.at[sb.at[j]],
                                  rb.at[pl.ds(j * 128, 128)], gsem).wait()

    def _fire_scatters(rb, db):
        for j in range(_C):
            pltpu.async_copy(rb.at[pl.ds(j * 128, 128)],
                             acc.at[db.at[j]], ssem, add=True)

    def _drain_scatters(rb, db):
        for j in range(_C):
            pltpu.make_async_copy(rb.at[pl.ds(j * 128, 128)],
                                  acc.at[db.at[j]], ssem).wait()

    bufs = ((src_a, dst_a, rows_a), (src_b, dst_b, rows_b))

    _load_idx(0, src_a, dst_a)
    _fire_gathers(src_a, rows_a)
    _load_idx(1, src_b, dst_b)
    _remap(dst_a)
    _drain_gathers(src_a, rows_a)
    _fire_gathers(src_b, rows_b)
    _fire_scatters(rows_a, dst_a)

    def _pair(k, carry):
        g = 1 + 2 * k
        for i, (cur, nxt) in enumerate(((bufs[1], bufs[0]),
                                        (bufs[0], bufs[1]))):
            cs, cd, cr = cur
            ns, nd, nr = nxt
            gg = g + i
            _remap(cd)
            _drain_scatters(nr, nd)
            _load_idx(gg + 1, ns, nd)
            _drain_gathers(cs, cr)
            _fire_gathers(ns, nr)
            _fire_scatters(cr, cd)
        return carry

    lax.fori_loop(0, (_ITERS - 2) // 2, _pair, 0)

    _remap(dst_b)
    _drain_scatters(rows_a, dst_a)
    _drain_gathers(src_b, rows_b)
    _fire_scatters(rows_b, dst_b)
    _drain_scatters(rows_b, dst_b)
    plsc.subcore_barrier()

    a0 = s * 3128

    @pl.when(s < 15)
    def _wb_main():
        pltpu.sync_copy(acc.at[pl.ds(a0, 3128)],
                        out_hbm.at[pl.ds(c * _HALF + a0, 3128)])

    @pl.when(s == 15)
    def _wb_tail():
        pltpu.sync_copy(acc.at[pl.ds(a0, 3080)],
                        out_hbm.at[pl.ds(c * _HALF + a0, 3080)])


@functools.cache
def _shift_call():
    return pl.kernel(
        _shift_body,
        out_type=jax.ShapeDtypeStruct((_N, _NC), jnp.bfloat16),
        mesh=plsc.VectorSubcoreMesh(core_axis_name="c", subcore_axis_name="s"),
        compiler_params=pltpu.CompilerParams(use_tc_tiling_on_sc=False),
        scratch_types=[
            pltpu.VMEM_SHARED((_ACC_ROWS, _NC), jnp.bfloat16),
            pltpu.VMEM((_C, 128), jnp.int32),
            pltpu.VMEM((_C, 128), jnp.int32),
            pltpu.VMEM((_C * 128, _NC), jnp.bfloat16),
            pltpu.VMEM((_C, 128), jnp.int32),
            pltpu.VMEM((_C, 128), jnp.int32),
            pltpu.VMEM((_C * 128, _NC), jnp.bfloat16),
            pltpu.SemaphoreType.DMA,
            pltpu.SemaphoreType.DMA,
        ],
    )


def _mlp_kernel(x_ref, w1_ref, b1_ref, w2_ref, b2_ref, o_ref):
    x = x_ref[...].astype(jnp.float32)
    h = jnp.dot(x, w1_ref[...], preferred_element_type=jnp.float32)
    h = jnp.maximum(h + b1_ref[...], 0.0)
    o_ref[...] = jnp.dot(h, w2_ref[...],
                         preferred_element_type=jnp.float32) + b2_ref[...]


def _mlp_res_kernel(x_ref, r_ref, w1_ref, b1_ref, w2_ref, b2_ref, o_ref):
    x = x_ref[...].astype(jnp.float32)
    h = jnp.dot(x, w1_ref[...], preferred_element_type=jnp.float32)
    h = jnp.maximum(h + b1_ref[...], 0.0)
    o_ref[...] = (r_ref[...] + jnp.dot(h, w2_ref[...],
                                       preferred_element_type=jnp.float32)
                  + b2_ref[...])


_ROW_BLK = 2000


def _mlp(x, p, residual=None):
    n, d_in = x.shape
    hdim = p["W1"].shape[1]
    d_out = p["W2"].shape[1]
    b1 = p["b1"].reshape(1, hdim)
    b2 = p["b2"].reshape(1, d_out)
    grid = (n // _ROW_BLK,)
    x_spec = pl.BlockSpec((_ROW_BLK, d_in), lambda i: (i, 0))
    w1_spec = pl.BlockSpec((d_in, hdim), lambda i: (0, 0))
    b1_spec = pl.BlockSpec((1, hdim), lambda i: (0, 0))
    w2_spec = pl.BlockSpec((hdim, d_out), lambda i: (0, 0))
    b2_spec = pl.BlockSpec((1, d_out), lambda i: (0, 0))
    o_spec = pl.BlockSpec((_ROW_BLK, d_out), lambda i: (i, 0))
    out_shape = jax.ShapeDtypeStruct((n, d_out), jnp.float32)
    if residual is None:
        return pl.pallas_call(
            _mlp_kernel, grid=grid,
            in_specs=[x_spec, w1_spec, b1_spec, w2_spec, b2_spec],
            out_specs=o_spec, out_shape=out_shape,
        )(x, p["W1"], b1, p["W2"], b2)
    r_spec = pl.BlockSpec((_ROW_BLK, d_out), lambda i: (i, 0))
    return pl.pallas_call(
        _mlp_res_kernel, grid=grid,
        in_specs=[x_spec, r_spec, w1_spec, b1_spec, w2_spec, b2_spec],
        out_specs=o_spec, out_shape=out_shape,
    )(x, residual, p["W1"], b1, p["W2"], b2)


def kernel(f, g, edge_index, params):
    src = edge_index[0]
    dst = edge_index[1]
    pad = _EP_ROWS * 128 - _E
    src_p = jnp.concatenate(
        [src, jnp.zeros((pad,), jnp.int32)]).reshape(_EP_ROWS, 128)
    dst_p = jnp.concatenate(
        [dst, jnp.full((pad,), _N, jnp.int32)]).reshape(_EP_ROWS, 128)
    zeros_stage = jnp.zeros((_C * 128, _NC), jnp.bfloat16)

    f1 = _mlp(f, params["readin_f"])
    g1 = _mlp(g, params["readin_g"])
    for l in range(2):
        fp = _shift_call()(f1.astype(jnp.bfloat16), src_p, dst_p, zeros_stage)
        gp = _shift_call()(g1.astype(jnp.bfloat16), src_p, dst_p, zeros_stage)
        f1n = _mlp(gp, params["convs"][l]["equi"], residual=f1)
        g1n = _mlp(fp, params["convs"][l]["inv"], residual=g1)
        f1, g1 = f1n, g1n
    return (_mlp(f1, params["readout_f"]), _mlp(g1, params["readout_g"]))


# speedup vs baseline: 1.1733x; 1.0085x over previous
import functools

import jax
import jax.numpy as jnp
from jax import lax
from jax.experimental import pallas as pl
from jax.experimental.pallas import tpu as pltpu
from jax.experimental.pallas import tpu_sc as plsc

_N = 100000
_NC = 32
_E = 1600000
_HALF = 50000
_ACC_ROWS = 50048
_EP_ROWS = 12800
_R_SUB = 800
_C = 16
_ITERS = _R_SUB // _C
_ZROWS = _ACC_ROWS // 16


def _shift_body(x_hbm, src_hbm, dst_hbm, zeros_hbm, out_hbm,
                acc, sa, da, ra, sb, db, rb, gsem, ssem, isem):
    c = lax.axis_index("c")
    s = lax.axis_index("s")

    pltpu.sync_copy(zeros_hbm, ra)
    z0 = s * _ZROWS
    pltpu.sync_copy(ra.at[pl.ds(0, 2048)], acc.at[pl.ds(z0, 2048)])
    pltpu.sync_copy(ra.at[pl.ds(0, 1080)], acc.at[pl.ds(z0 + 2048, 1080)])
    plsc.subcore_barrier()

    cbase = s * _ITERS

    def _fire_idx(g, sref, dref):
        pltpu.async_copy(src_hbm.at[cbase + g], sref, isem)
        pltpu.async_copy(dst_hbm.at[c].at[cbase + g], dref, isem)

    def _drain_idx(g, sref, dref):
        pltpu.make_async_copy(src_hbm.at[cbase + g], sref, isem).wait()
        pltpu.make_async_copy(dst_hbm.at[c].at[cbase + g], dref, isem).wait()

    def _fire_gathers(sref, rref):
        pltpu.async_copy(x_hbm.at[sref], rref, gsem)

    def _drain_gathers(sref, rref):
        pltpu.make_async_copy(x_hbm.at[sref], rref, gsem).wait()

    def _fire_scatters(rref, dref):
        pltpu.async_copy(rref, acc.at[dref], ssem, add=True)

    def _drain_scatters(rref, dref):
        pltpu.make_async_copy(rref, acc.at[dref], ssem).wait()

    bufs = ((sa, da, ra), (sb, db, rb))

    pltpu.sync_copy(src_hbm.at[cbase], sa)
    pltpu.sync_copy(dst_hbm.at[c].at[cbase], da)
    _fire_gathers(sa, ra)
    _fire_idx(1, sb, db)
    _drain_gathers(sa, ra)
    _drain_idx(1, sb, db)
    _fire_gathers(sb, rb)
    _fire_scatters(ra, da)

    def _pair(k, carry):
        g = 1 + 2 * k
        for i, (cur, nxt) in enumerate(((bufs[1], bufs[0]),
                                        (bufs[0], bufs[1]))):
            cs, cd, cr = cur
            ns, nd, nr = nxt
            gg = g + i
            _drain_scatters(nr, nd)
            _fire_idx(gg + 1, ns, nd)
            _drain_gathers(cs, cr)
            _drain_idx(gg + 1, ns, nd)
            _fire_gathers(ns, nr)
            _fire_scatters(cr, cd)
        return carry

    lax.fori_loop(0, (_ITERS - 2) // 2, _pair, 0)

    _drain_scatters(ra, da)
    _drain_gathers(sb, rb)
    _fire_scatters(rb, db)
    _drain_scatters(rb, db)
    plsc.subcore_barrier()

    a0 = s * 3128

    @pl.when(s < 15)
    def _wb_main():
        pltpu.sync_copy(acc.at[pl.ds(a0, 3128)],
                        out_hbm.at[pl.ds(c * _HALF + a0, 3128)])

    @pl.when(s == 15)
    def _wb_tail():
        pltpu.sync_copy(acc.at[pl.ds(a0, 3080)],
                        out_hbm.at[pl.ds(c * _HALF + a0, 3080)])


@functools.cache
def _shift_call():
    return pl.kernel(
        _shift_body,
        out_type=jax.ShapeDtypeStruct((_N, _NC), jnp.bfloat16),
        mesh=plsc.VectorSubcoreMesh(core_axis_name="c", subcore_axis_name="s"),
        compiler_params=pltpu.CompilerParams(use_tc_tiling_on_sc=False),
        scratch_types=[
            pltpu.VMEM_SHARED((_ACC_ROWS, _NC), jnp.bfloat16),
            pltpu.VMEM((_C * 128,), jnp.int32),
            pltpu.VMEM((_C * 128,), jnp.int32),
            pltpu.VMEM((_C * 128, _NC), jnp.bfloat16),
            pltpu.VMEM((_C * 128,), jnp.int32),
            pltpu.VMEM((_C * 128,), jnp.int32),
            pltpu.VMEM((_C * 128, _NC), jnp.bfloat16),
            pltpu.SemaphoreType.DMA,
            pltpu.SemaphoreType.DMA,
            pltpu.SemaphoreType.DMA,
        ],
    )


def _mlp_kernel(x_ref, w1_ref, b1_ref, w2_ref, b2_ref, o_ref):
    x = x_ref[...].astype(jnp.float32)
    h = jnp.dot(x, w1_ref[...], preferred_element_type=jnp.float32)
    h = jnp.maximum(h + b1_ref[...], 0.0)
    o_ref[...] = jnp.dot(h, w2_ref[...],
                         preferred_element_type=jnp.float32) + b2_ref[...]


def _mlp_res_kernel(x_ref, r_ref, w1_ref, b1_ref, w2_ref, b2_ref, o_ref):
    x = x_ref[...].astype(jnp.float32)
    h = jnp.dot(x, w1_ref[...], preferred_element_type=jnp.float32)
    h = jnp.maximum(h + b1_ref[...], 0.0)
    o_ref[...] = (r_ref[...] + jnp.dot(h, w2_ref[...],
                                       preferred_element_type=jnp.float32)
                  + b2_ref[...])


_ROW_BLK = 2000


def _mlp(x, p, residual=None):
    n, d_in = x.shape
    hdim = p["W1"].shape[1]
    d_out = p["W2"].shape[1]
    b1 = p["b1"].reshape(1, hdim)
    b2 = p["b2"].reshape(1, d_out)
    grid = (n // _ROW_BLK,)
    x_spec = pl.BlockSpec((_ROW_BLK, d_in), lambda i: (i, 0))
    w1_spec = pl.BlockSpec((d_in, hdim), lambda i: (0, 0))
    b1_spec = pl.BlockSpec((1, hdim), lambda i: (0, 0))
    w2_spec = pl.BlockSpec((hdim, d_out), lambda i: (0, 0))
    b2_spec = pl.BlockSpec((1, d_out), lambda i: (0, 0))
    o_spec = pl.BlockSpec((_ROW_BLK, d_out), lambda i: (i, 0))
    out_shape = jax.ShapeDtypeStruct((n, d_out), jnp.float32)
    if residual is None:
        return pl.pallas_call(
            _mlp_kernel, grid=grid,
            in_specs=[x_spec, w1_spec, b1_spec, w2_spec, b2_spec],
            out_specs=o_spec, out_shape=out_shape,
        )(x, p["W1"], b1, p["W2"], b2)
    r_spec = pl.BlockSpec((_ROW_BLK, d_out), lambda i: (i, 0))
    return pl.pallas_call(
        _mlp_res_kernel, grid=grid,
        in_specs=[x_spec, r_spec, w1_spec, b1_spec, w2_spec, b2_spec],
        out_specs=o_spec, out_shape=out_shape,
    )(x, residual, p["W1"], b1, p["W2"], b2)


def kernel(f, g, edge_index, params):
    src = edge_index[0]
    dst = edge_index[1]
    pad = _EP_ROWS * 128 - _E
    src_p = jnp.concatenate([src, jnp.zeros((pad,), jnp.int32)])
    dst_p = jnp.concatenate([dst, jnp.full((pad,), _N, jnp.int32)])
    garb = _HALF + (jnp.arange(_EP_ROWS * 128, dtype=jnp.int32) & 31)
    dst0 = jnp.where(dst_p < _HALF, dst_p, garb)
    dst1 = jnp.where(dst_p >= _HALF, dst_p - _HALF, garb)
    src_arr = src_p.reshape(-1, _C * 128)
    dst_arr = jnp.stack([dst0.reshape(-1, _C * 128),
                         dst1.reshape(-1, _C * 128)])
    zeros_stage = jnp.zeros((_C * 128, _NC), jnp.bfloat16)

    f1 = _mlp(f, params["readin_f"])
    g1 = _mlp(g, params["readin_g"])
    for l in range(2):
        fp = _shift_call()(f1.astype(jnp.bfloat16), src_arr, dst_arr,
                           zeros_stage)
        gp = _shift_call()(g1.astype(jnp.bfloat16), src_arr, dst_arr,
                           zeros_stage)
        f1n = _mlp(gp, params["convs"][l]["equi"], residual=f1)
        g1n = _mlp(fp, params["convs"][l]["inv"], residual=g1)
        f1, g1 = f1n, g1n
    return (_mlp(f1, params["readout_f"]), _mlp(g1, params["readout_g"]))


# speedup vs baseline: 1.3804x; 1.1765x over previous
import functools

import jax
import jax.numpy as jnp
from jax import lax
from jax.experimental import pallas as pl
from jax.experimental.pallas import tpu as pltpu
from jax.experimental.pallas import tpu_sc as plsc

_N = 100000
_NC = 32
_E = 1600000
_ACC_ROWS = 100048
_EP_ROWS = 12800
_C = 4
_CHUNK = _C * 128
_ITERS = 100
_ZMAIN = 6256


def _shift_body(x_hbm, src_hbm, dst_hbm, zeros_hbm, out_hbm,
                acc, sa, da, ra, sb, db, rb, gsem, ssem, isem):
    c = lax.axis_index("c")
    s = lax.axis_index("s")

    pltpu.sync_copy(zeros_hbm, ra)
    z0 = s * _ZMAIN
    for k in range(12):
        pltpu.sync_copy(ra.at[pl.ds(0, 512)],
                        acc.at[pl.ds(z0 + k * 512, 512)])

    @pl.when(s < 15)
    def _z_main():
        pltpu.sync_copy(ra.at[pl.ds(0, 112)], acc.at[pl.ds(z0 + 6144, 112)])

    @pl.when(s == 15)
    def _z_tail():
        pltpu.sync_copy(ra.at[pl.ds(0, 64)], acc.at[pl.ds(z0 + 6144, 64)])

    plsc.subcore_barrier()

    cbase = (c * 16 + s) * _ITERS

    def _fire_idx(g, sref, dref):
        pltpu.async_copy(src_hbm.at[cbase + g], sref, isem)
        pltpu.async_copy(dst_hbm.at[cbase + g], dref, isem)

    def _drain_idx(g, sref, dref):
        pltpu.make_async_copy(src_hbm.at[cbase + g], sref, isem).wait()
        pltpu.make_async_copy(dst_hbm.at[cbase + g], dref, isem).wait()

    def _fire_gathers(sref, rref):
        pltpu.async_copy(x_hbm.at[sref], rref, gsem)

    def _drain_gathers(sref, rref):
        pltpu.make_async_copy(x_hbm.at[sref], rref, gsem).wait()

    def _fire_scatters(rref, dref):
        pltpu.async_copy(rref, acc.at[dref], ssem, add=True)

    def _drain_scatters(rref, dref):
        pltpu.make_async_copy(rref, acc.at[dref], ssem).wait()

    bufs = ((sa, da, ra), (sb, db, rb))

    pltpu.sync_copy(src_hbm.at[cbase], sa)
    pltpu.sync_copy(dst_hbm.at[cbase], da)
    _fire_gathers(sa, ra)
    _fire_idx(1, sb, db)
    _drain_gathers(sa, ra)
    _drain_idx(1, sb, db)
    _fire_gathers(sb, rb)
    _fire_scatters(ra, da)

    def _pair(k, carry):
        g = 1 + 2 * k
        for i, (cur, nxt) in enumerate(((bufs[1], bufs[0]),
                                        (bufs[0], bufs[1]))):
            cs, cd, cr = cur
            ns, nd, nr = nxt
            gg = g + i
            _drain_scatters(nr, nd)
            _fire_idx(gg + 1, ns, nd)
            _drain_gathers(cs, cr)
            _drain_idx(gg + 1, ns, nd)
            _fire_gathers(ns, nr)
            _fire_scatters(cr, cd)
        return carry

    lax.fori_loop(0, (_ITERS - 2) // 2, _pair, 0)

    _drain_scatters(ra, da)
    _drain_gathers(sb, rb)
    _fire_scatters(rb, db)
    _drain_scatters(rb, db)
    plsc.subcore_barrier()

    a0 = s * _ZMAIN

    @pl.when(s < 15)
    def _wb_main():
        pltpu.sync_copy(acc.at[pl.ds(a0, 6256)],
                        out_hbm.at[c].at[pl.ds(a0, 6256)])

    @pl.when(s == 15)
    def _wb_tail():
        pltpu.sync_copy(acc.at[pl.ds(a0, 6160)],
                        out_hbm.at[c].at[pl.ds(a0, 6160)])


@functools.cache
def _shift_call():
    return pl.kernel(
        _shift_body,
        out_type=jax.ShapeDtypeStruct((2, _N, _NC), jnp.bfloat16),
        mesh=plsc.VectorSubcoreMesh(core_axis_name="c", subcore_axis_name="s"),
        compiler_params=pltpu.CompilerParams(use_tc_tiling_on_sc=False),
        scratch_types=[
            pltpu.VMEM_SHARED((_ACC_ROWS, _NC), jnp.bfloat16),
            pltpu.VMEM((_CHUNK,), jnp.int32),
            pltpu.VMEM((_CHUNK,), jnp.int32),
            pltpu.VMEM((_CHUNK, _NC), jnp.bfloat16),
            pltpu.VMEM((_CHUNK,), jnp.int32),
            pltpu.VMEM((_CHUNK,), jnp.int32),
            pltpu.VMEM((_CHUNK, _NC), jnp.bfloat16),
            pltpu.SemaphoreType.DMA,
            pltpu.SemaphoreType.DMA,
            pltpu.SemaphoreType.DMA,
        ],
    )


def _mlp_kernel(x_ref, w1_ref, b1_ref, w2_ref, b2_ref, o_ref):
    x = x_ref[...].astype(jnp.float32)
    h = jnp.dot(x, w1_ref[...], preferred_element_type=jnp.float32)
    h = jnp.maximum(h + b1_ref[...], 0.0)
    o_ref[...] = jnp.dot(h, w2_ref[...],
                         preferred_element_type=jnp.float32) + b2_ref[...]


def _mlp_res2_kernel(x0_ref, x1_ref, r_ref, w1_ref, b1_ref, w2_ref, b2_ref,
                     o_ref):
    x = x0_ref[...].astype(jnp.float32) + x1_ref[...].astype(jnp.float32)
    h = jnp.dot(x, w1_ref[...], preferred_element_type=jnp.float32)
    h = jnp.maximum(h + b1_ref[...], 0.0)
    o_ref[...] = (r_ref[...] + jnp.dot(h, w2_ref[...],
                                       preferred_element_type=jnp.float32)
                  + b2_ref[...])


_ROW_BLK = 2000


def _mlp(x, p):
    n, d_in = x.shape
    hdim = p["W1"].shape[1]
    d_out = p["W2"].shape[1]
    b1 = p["b1"].reshape(1, hdim)
    b2 = p["b2"].reshape(1, d_out)
    grid = (n // _ROW_BLK,)
    return pl.pallas_call(
        _mlp_kernel, grid=grid,
        in_specs=[pl.BlockSpec((_ROW_BLK, d_in), lambda i: (i, 0)),
                  pl.BlockSpec((d_in, hdim), lambda i: (0, 0)),
                  pl.BlockSpec((1, hdim), lambda i: (0, 0)),
                  pl.BlockSpec((hdim, d_out), lambda i: (0, 0)),
                  pl.BlockSpec((1, d_out), lambda i: (0, 0))],
        out_specs=pl.BlockSpec((_ROW_BLK, d_out), lambda i: (i, 0)),
        out_shape=jax.ShapeDtypeStruct((n, d_out), jnp.float32),
    )(x, p["W1"], b1, p["W2"], b2)


def _mlp_res2(x0, x1, p, residual):
    n, d_in = x0.shape
    hdim = p["W1"].shape[1]
    d_out = p["W2"].shape[1]
    b1 = p["b1"].reshape(1, hdim)
    b2 = p["b2"].reshape(1, d_out)
    grid = (n // _ROW_BLK,)
    x_spec = pl.BlockSpec((_ROW_BLK, d_in), lambda i: (i, 0))
    return pl.pallas_call(
        _mlp_res2_kernel, grid=grid,
        in_specs=[x_spec, x_spec,
                  pl.BlockSpec((_ROW_BLK, d_out), lambda i: (i, 0)),
                  pl.BlockSpec((d_in, hdim), lambda i: (0, 0)),
                  pl.BlockSpec((1, hdim), lambda i: (0, 0)),
                  pl.BlockSpec((hdim, d_out), lambda i: (0, 0)),
                  pl.BlockSpec((1, d_out), lambda i: (0, 0))],
        out_specs=pl.BlockSpec((_ROW_BLK, d_out), lambda i: (i, 0)),
        out_shape=jax.ShapeDtypeStruct((n, d_out), jnp.float32),
    )(x0, x1, residual, p["W1"], b1, p["W2"], b2)


def kernel(f, g, edge_index, params):
    src = edge_index[0]
    dst = edge_index[1]
    pad = _EP_ROWS * 128 - _E
    src_p = jnp.concatenate([src, jnp.zeros((pad,), jnp.int32)])
    dst_p = jnp.concatenate([dst, jnp.full((pad,), _N, jnp.int32)])
    garb = _N + (jnp.arange(_EP_ROWS * 128, dtype=jnp.int32) & 31)
    dst_p = jnp.where(dst_p < _N, dst_p, garb)
    src_arr = src_p.reshape(-1, _CHUNK)
    dst_arr = dst_p.reshape(-1, _CHUNK)
    zeros_stage = jnp.zeros((_CHUNK, _NC), jnp.bfloat16)

    f1 = _mlp(f, params["readin_f"])
    g1 = _mlp(g, params["readin_g"])
    for l in range(2):
        fp = _shift_call()(f1.astype(jnp.bfloat16), src_arr, dst_arr,
                           zeros_stage)
        gp = _shift_call()(g1.astype(jnp.bfloat16), src_arr, dst_arr,
                           zeros_stage)
        f1n = _mlp_res2(gp[0], gp[1], params["convs"][l]["equi"], f1)
        g1n = _mlp_res2(fp[0], fp[1], params["convs"][l]["inv"], g1)
        f1, g1 = f1n, g1n
    return (_mlp(f1, params["readout_f"]), _mlp(g1, params["readout_g"]))


# speedup vs baseline: 1.4123x; 1.0231x over previous
import functools

import jax
import jax.numpy as jnp
from jax import lax
from jax.experimental import pallas as pl
from jax.experimental.pallas import tpu as pltpu
from jax.experimental.pallas import tpu_sc as plsc

_N = 100000
_NC = 32
_E = 1600000
_ACC_ROWS = 100048
_EP_ROWS = 12800
_C = 4
_CHUNK = _C * 128
_ITERS = 100
_ZMAIN = 6256


def _shift_body(x_hbm, src_hbm, dst_hbm, zeros_hbm, out_hbm,
                acc, sa, da, ra, sb, db, rb, gsem, ssem, isem):
    c = lax.axis_index("c")
    s = lax.axis_index("s")

    pltpu.sync_copy(zeros_hbm, ra)
    z0 = s * _ZMAIN
    for k in range(12):
        pltpu.sync_copy(ra.at[pl.ds(0, 512)],
                        acc.at[pl.ds(z0 + k * 512, 512)])

    @pl.when(s < 15)
    def _z_main():
        pltpu.sync_copy(ra.at[pl.ds(0, 112)], acc.at[pl.ds(z0 + 6144, 112)])

    @pl.when(s == 15)
    def _z_tail():
        pltpu.sync_copy(ra.at[pl.ds(0, 64)], acc.at[pl.ds(z0 + 6144, 64)])

    plsc.subcore_barrier()

    cbase = (c * 16 + s) * _ITERS

    def _fire_idx(g, sref, dref):
        pltpu.async_copy(src_hbm.at[cbase + g], sref, isem)
        pltpu.async_copy(dst_hbm.at[cbase + g], dref, isem)

    def _drain_idx(g, sref, dref):
        pltpu.make_async_copy(src_hbm.at[cbase + g], sref, isem).wait()
        pltpu.make_async_copy(dst_hbm.at[cbase + g], dref, isem).wait()

    def _fire_gathers(sref, rref):
        pltpu.async_copy(x_hbm.at[sref], rref, gsem)

    def _drain_gathers(sref, rref):
        pltpu.make_async_copy(x_hbm.at[sref], rref, gsem).wait()

    def _fire_scatters(rref, dref):
        pltpu.async_copy(rref, acc.at[dref], ssem, add=True)

    def _drain_scatters(rref, dref):
        pltpu.make_async_copy(rref, acc.at[dref], ssem).wait()

    bufs = ((sa, da, ra), (sb, db, rb))

    pltpu.sync_copy(src_hbm.at[cbase], sa)
    pltpu.sync_copy(dst_hbm.at[cbase], da)
    _fire_gathers(sa, ra)
    _fire_idx(1, sb, db)
    _drain_gathers(sa, ra)
    _drain_idx(1, sb, db)
    _fire_gathers(sb, rb)
    _fire_scatters(ra, da)

    def _pair(k, carry):
        g = 1 + 2 * k
        for i, (cur, nxt) in enumerate(((bufs[1], bufs[0]),
                                        (bufs[0], bufs[1]))):
            cs, cd, cr = cur
            ns, nd, nr = nxt
            gg = g + i
            _drain_scatters(nr, nd)
            _fire_idx(gg + 1, ns, nd)
            _drain_gathers(cs, cr)
            _drain_idx(gg + 1, ns, nd)
            _fire_gathers(ns, nr)
            _fire_scatters(cr, cd)
        return carry

    lax.fori_loop(0, (_ITERS - 2) // 2, _pair, 0)

    _drain_scatters(ra, da)
    _drain_gathers(sb, rb)
    _fire_scatters(rb, db)
    _drain_scatters(rb, db)
    plsc.subcore_barrier()

    a0 = s * _ZMAIN

    @pl.when(s < 15)
    def _wb_main():
        pltpu.sync_copy(acc.at[pl.ds(a0, 6256)],
                        out_hbm.at[c].at[pl.ds(a0, 6256)])

    @pl.when(s == 15)
    def _wb_tail():
        pltpu.sync_copy(acc.at[pl.ds(a0, 6160)],
                        out_hbm.at[c].at[pl.ds(a0, 6160)])


@functools.cache
def _shift_call():
    return pl.kernel(
        _shift_body,
        out_type=jax.ShapeDtypeStruct((2, _N, _NC), jnp.bfloat16),
        mesh=plsc.VectorSubcoreMesh(core_axis_name="c", subcore_axis_name="s"),
        compiler_params=pltpu.CompilerParams(use_tc_tiling_on_sc=False),
        scratch_types=[
            pltpu.VMEM_SHARED((_ACC_ROWS, _NC), jnp.bfloat16),
            pltpu.VMEM((_CHUNK,), jnp.int32),
            pltpu.VMEM((_CHUNK,), jnp.int32),
            pltpu.VMEM((_CHUNK, _NC), jnp.bfloat16),
            pltpu.VMEM((_CHUNK,), jnp.int32),
            pltpu.VMEM((_CHUNK,), jnp.int32),
            pltpu.VMEM((_CHUNK, _NC), jnp.bfloat16),
            pltpu.SemaphoreType.DMA,
            pltpu.SemaphoreType.DMA,
            pltpu.SemaphoreType.DMA,
        ],
    )


def _mlp_kernel(x_ref, w1_ref, b1_ref, w2_ref, b2_ref, o_ref):
    x = x_ref[...].astype(jnp.bfloat16)
    h = jnp.dot(x, w1_ref[...], preferred_element_type=jnp.float32)
    h = jnp.maximum(h + b1_ref[...], 0.0).astype(jnp.bfloat16)
    o_ref[...] = jnp.dot(h, w2_ref[...],
                         preferred_element_type=jnp.float32) + b2_ref[...]


def _mlp_res2_kernel(x0_ref, x1_ref, r_ref, w1_ref, b1_ref, w2_ref, b2_ref,
                     o_ref):
    x = (x0_ref[...].astype(jnp.float32)
         + x1_ref[...].astype(jnp.float32)).astype(jnp.bfloat16)
    h = jnp.dot(x, w1_ref[...], preferred_element_type=jnp.float32)
    h = jnp.maximum(h + b1_ref[...], 0.0).astype(jnp.bfloat16)
    o_ref[...] = (r_ref[...] + jnp.dot(h, w2_ref[...],
                                       preferred_element_type=jnp.float32)
                  + b2_ref[...])


_ROW_BLK = 2000


def _mlp(x, p):
    n, d_in = x.shape
    hdim = p["W1"].shape[1]
    d_out = p["W2"].shape[1]
    b1 = p["b1"].reshape(1, hdim)
    b2 = p["b2"].reshape(1, d_out)
    grid = (n // _ROW_BLK,)
    return pl.pallas_call(
        _mlp_kernel, grid=grid,
        in_specs=[pl.BlockSpec((_ROW_BLK, d_in), lambda i: (i, 0)),
                  pl.BlockSpec((d_in, hdim), lambda i: (0, 0)),
                  pl.BlockSpec((1, hdim), lambda i: (0, 0)),
                  pl.BlockSpec((hdim, d_out), lambda i: (0, 0)),
                  pl.BlockSpec((1, d_out), lambda i: (0, 0))],
        out_specs=pl.BlockSpec((_ROW_BLK, d_out), lambda i: (i, 0)),
        out_shape=jax.ShapeDtypeStruct((n, d_out), jnp.float32),
    )(x, p["W1"].astype(jnp.bfloat16), b1, p["W2"].astype(jnp.bfloat16), b2)


def _mlp_res2(x0, x1, p, residual):
    n, d_in = x0.shape
    hdim = p["W1"].shape[1]
    d_out = p["W2"].shape[1]
    b1 = p["b1"].reshape(1, hdim)
    b2 = p["b2"].reshape(1, d_out)
    grid = (n // _ROW_BLK,)
    x_spec = pl.BlockSpec((_ROW_BLK, d_in), lambda i: (i, 0))
    return pl.pallas_call(
        _mlp_res2_kernel, grid=grid,
        in_specs=[x_spec, x_spec,
                  pl.BlockSpec((_ROW_BLK, d_out), lambda i: (i, 0)),
                  pl.BlockSpec((d_in, hdim), lambda i: (0, 0)),
                  pl.BlockSpec((1, hdim), lambda i: (0, 0)),
                  pl.BlockSpec((hdim, d_out), lambda i: (0, 0)),
                  pl.BlockSpec((1, d_out), lambda i: (0, 0))],
        out_specs=pl.BlockSpec((_ROW_BLK, d_out), lambda i: (i, 0)),
        out_shape=jax.ShapeDtypeStruct((n, d_out), jnp.float32),
    )(x0, x1, residual, p["W1"].astype(jnp.bfloat16), b1,
      p["W2"].astype(jnp.bfloat16), b2)


def kernel(f, g, edge_index, params):
    src = edge_index[0]
    dst = edge_index[1]
    pad = _EP_ROWS * 128 - _E
    src_p = jnp.concatenate([src, jnp.zeros((pad,), jnp.int32)])
    dst_p = jnp.concatenate([dst, jnp.full((pad,), _N, jnp.int32)])
    garb = _N + (jnp.arange(_EP_ROWS * 128, dtype=jnp.int32) & 31)
    dst_p = jnp.where(dst_p < _N, dst_p, garb)
    src_arr = src_p.reshape(-1, _CHUNK)
    dst_arr = dst_p.reshape(-1, _CHUNK)
    zeros_stage = jnp.zeros((_CHUNK, _NC), jnp.bfloat16)

    f1 = _mlp(f, params["readin_f"])
    g1 = _mlp(g, params["readin_g"])
    for l in range(2):
        fp = _shift_call()(f1.astype(jnp.bfloat16), src_arr, dst_arr,
                           zeros_stage)
        gp = _shift_call()(g1.astype(jnp.bfloat16), src_arr, dst_arr,
                           zeros_stage)
        f1n = _mlp_res2(gp[0], gp[1], params["convs"][l]["equi"], f1)
        g1n = _mlp_res2(fp[0], fp[1], params["convs"][l]["inv"], g1)
        f1, g1 = f1n, g1n
    return (_mlp(f1, params["readout_f"]), _mlp(g1, params["readout_g"]))
